# baseline (speedup 1.0000x reference)
import functools

import jax
import jax.numpy as jnp
from jax import lax
from jax.experimental import pallas as pl
from jax.experimental.pallas import tpu as pltpu
from jax.experimental.pallas import tpu_sc as plsc

N_NODES = 10000
N_EDGES = 320000
D = 128

CHUNK = 96
NCHUNKS = -(-N_EDGES // CHUNK)
PAD_EDGES = NCHUNKS * CHUNK
NCORES = 2
NSUB = 16
NWORKERS = NCORES * NSUB
ITERS = -(-NCHUNKS // NWORKERS)
TRIPLES = ITERS // 3
RCHUNK = 80
NRCHUNKS = N_NODES // RCHUNK
RITERS = -(-NRCHUNKS // NSUB)


def _mm_body(x_ref, w_ref, o_ref):
    o_ref[...] = jnp.dot(x_ref[...], w_ref[...],
                         preferred_element_type=jnp.float32)


def _matmul(x, w):
    return pl.pallas_call(
        _mm_body,
        grid=(10,),
        in_specs=[
            pl.BlockSpec((N_NODES // 10, D), lambda i: (i, 0)),
            pl.BlockSpec((D, D), lambda i: (0, 0)),
        ],
        out_specs=pl.BlockSpec((N_NODES // 10, D), lambda i: (i, 0)),
        out_shape=jax.ShapeDtypeStruct((N_NODES, D), jnp.float32),
    )(x, w)


def _comb_body(p_ref, b_ref, o_ref):
    o_ref[...] = p_ref[0] + p_ref[1] + b_ref[...]


def _combine(partials, b):
    return pl.pallas_call(
        _comb_body,
        grid=(10,),
        in_specs=[
            pl.BlockSpec((2, N_NODES // 10, D), lambda i: (0, i, 0)),
            pl.BlockSpec((1, D), lambda i: (0, 0)),
        ],
        out_specs=pl.BlockSpec((N_NODES // 10, D), lambda i: (i, 0)),
        out_shape=jax.ShapeDtypeStruct((N_NODES, D), jnp.float32),
    )(partials, b)


def _scale_rows(rows_ref, vbuf):
    @pl.loop(0, CHUNK // 16)
    def _(eb):
        v16 = vbuf[pl.ds(0, 1), pl.ds(eb * 16, 16)]
        for j in range(16):
            v = v16[0, j]
            for g in range(D // 16):
                sl = (pl.ds(eb * 16 + j, 1), pl.ds(g * 16, 16))
                rows_ref[sl] = rows_ref[sl] * v


def _spmm(hidden, eidx, vals):
    mesh = plsc.VectorSubcoreMesh(core_axis_name="core",
                                  subcore_axis_name="subcore")

    @functools.partial(
        pl.kernel,
        out_type=jax.ShapeDtypeStruct((NCORES, N_NODES, D), jnp.float32),
        mesh=mesh,
        scratch_types=[
            pltpu.VMEM((2, CHUNK), jnp.int32),
            pltpu.VMEM((2, CHUNK), jnp.int32),
            pltpu.VMEM((2, CHUNK), jnp.int32),
            pltpu.VMEM((1, CHUNK), jnp.float32),
            pltpu.VMEM((1, CHUNK), jnp.float32),
            pltpu.VMEM((1, CHUNK), jnp.float32),
            pltpu.VMEM((CHUNK, D), jnp.float32),
            pltpu.VMEM((CHUNK, D), jnp.float32),
            pltpu.VMEM((CHUNK, D), jnp.float32),
            pltpu.VMEM_SHARED((N_NODES, D), jnp.float32),
            pltpu.SemaphoreType.DMA,
            pltpu.SemaphoreType.DMA,
            pltpu.SemaphoreType.DMA,
            pltpu.SemaphoreType.DMA,
            pltpu.SemaphoreType.DMA,
            pltpu.SemaphoreType.DMA,
        ],
    )
    def spmm_kernel(hid_hbm, eidx_hbm, val_hbm, part_hbm,
                    eb0, eb1, eb2, vb0, vb1, vb2, rw0, rw1, rw2, acc,
                    g0, g1, g2, s0, s1, s2):
        slots = ((eb0, vb0, rw0, g0, s0),
                 (eb1, vb1, rw1, g1, s1),
                 (eb2, vb2, rw2, g2, s2))
        cid = lax.axis_index("core")
        tid = lax.axis_index("subcore")
        wid = tid * NCORES + cid

        @pl.loop(0, RCHUNK)
        def _(r):
            for g in range(D // 16):
                rw0[pl.ds(r, 1), pl.ds(g * 16, 16)] = jnp.zeros(
                    (1, 16), jnp.float32)

        zsrc = rw0.at[pl.ds(0, RCHUNK)]
        for k in range(RITERS):
            rc = k * NSUB + tid

            @pl.when(rc < NRCHUNKS)
            def _():
                pltpu.sync_copy(zsrc, acc.at[pl.ds(rc * RCHUNK, RCHUNK)])
        plsc.subcore_barrier()

        pltpu.sync_copy(eidx_hbm.at[wid], eb0)
        pltpu.sync_copy(val_hbm.at[pl.ds(wid, 1)], vb0)
        pltpu.async_copy(hid_hbm.at[eb0.at[0]], rw0, g0)

        @pl.loop(0, TRIPLES)
        def _(h):
            for p in range(3):
                ebuf, vbuf, rows_v, gsem, ssem = slots[p]
                ebn, vbn, rwn, gsn, ssn = slots[(p + 1) % 3]
                i = h * 3 + p
                chunk = i * NWORKERS + wid

                @pl.when(chunk < NCHUNKS)
                def _():
                    pltpu.make_async_copy(
                        hid_hbm.at[ebuf.at[0]], rows_v, gsem).wait()
                    _scale_rows(rows_v, vbuf)
                    pltpu.async_copy(
                        rows_v, acc.at[ebuf.at[1]], ssem, add=True)

                nchunk = (i + 1) * NWORKERS + wid

                @pl.when(nchunk < NCHUNKS)
                def _():
                    @pl.when(i + 1 >= 3)
                    def _():
                        pltpu.make_async_copy(
                            rwn, acc.at[ebn.at[1]], ssn).wait()

                    pltpu.sync_copy(eidx_hbm.at[nchunk], ebn)
                    pltpu.sync_copy(val_hbm.at[pl.ds(nchunk, 1)], vbn)
                    pltpu.async_copy(hid_hbm.at[ebn.at[0]], rwn, gsn)

        for p in range(3):
            ebuf, vbuf, rows_v, gsem, ssem = slots[p]
            pltpu.make_async_copy(rows_v, acc.at[ebuf.at[1]], ssem).wait()
        plsc.subcore_barrier()

        for k in range(RITERS):
            rc = k * NSUB + tid

            @pl.when(rc < NRCHUNKS)
            def _():
                pltpu.sync_copy(
                    acc.at[pl.ds(rc * RCHUNK, RCHUNK)],
                    part_hbm.at[cid, pl.ds(rc * RCHUNK, RCHUNK)])

    return spmm_kernel(hidden, eidx, vals)


def _pad(a, dtype):
    a = a.astype(dtype)
    pad = jnp.zeros((PAD_EDGES - N_EDGES,), dtype)
    return jnp.concatenate([a, pad]).reshape(NCHUNKS, CHUNK)


def kernel(input, edge_index, edge_vals, W, b):
    hidden = _matmul(input, W)
    dst = _pad(edge_index[0], jnp.int32)
    src = _pad(edge_index[1], jnp.int32)
    eidx = jnp.stack([src, dst], axis=1)
    vals = _pad(edge_vals, jnp.float32)
    partials = _spmm(hidden, eidx, vals)
    return _combine(partials, b)


# speedup vs baseline: 1.5982x; 1.5982x over previous
import dataclasses
import functools

import jax
import jax.numpy as jnp
from jax import lax
from jax.experimental import pallas as pl
from jax.experimental.pallas import tpu as pltpu
from jax.experimental.pallas import tpu_sc as plsc

N_NODES = 10000
N_EDGES = 320000
D = 128

CHUNK = 128
NCHUNKS = N_EDGES // CHUNK
NCORES = 2
NSUB = 16
NWORKERS = NCORES * NSUB
ITERS = -(-NCHUNKS // NWORKERS)
QUADS = (ITERS + 3) // 4
RCHUNK = 80
NRCHUNKS = N_NODES // RCHUNK
RITERS = -(-NRCHUNKS // NSUB)


def _mm_body(x_ref, w_ref, o_ref):
    o_ref[...] = jnp.dot(x_ref[...], w_ref[...],
                         preferred_element_type=jnp.float32)


def _matmul(x, w):
    return pl.pallas_call(
        _mm_body,
        grid=(10,),
        in_specs=[
            pl.BlockSpec((N_NODES // 10, D), lambda i: (i, 0)),
            pl.BlockSpec((D, D), lambda i: (0, 0)),
        ],
        out_specs=pl.BlockSpec((N_NODES // 10, D), lambda i: (i, 0)),
        out_shape=jax.ShapeDtypeStruct((N_NODES, D), jnp.float32),
    )(x, w)


def _comb_body(p_ref, b_ref, o_ref):
    o_ref[...] = p_ref[0] + p_ref[1] + b_ref[...]


def _combine(partials, b):
    return pl.pallas_call(
        _comb_body,
        grid=(10,),
        in_specs=[
            pl.BlockSpec((2, N_NODES // 10, D), lambda i: (0, i, 0)),
            pl.BlockSpec((1, D), lambda i: (0, 0)),
        ],
        out_specs=pl.BlockSpec((N_NODES // 10, D), lambda i: (i, 0)),
        out_shape=jax.ShapeDtypeStruct((N_NODES, D), jnp.float32),
    )(partials, b)


def _scale_rows(rows_ref, vbuf):
    @pl.loop(0, CHUNK // 16)
    def _(eb):
        v16 = vbuf[pl.ds(0, 1), pl.ds(eb * 16, 16)]
        for j in range(16):
            v = v16[0, j]
            for g in range(D // 16):
                sl = (pl.ds(eb * 16 + j, 1), pl.ds(g * 16, 16))
                rows_ref[sl] = rows_ref[sl] * v


def _spmm(hidden, eidx, vals):
    mesh = plsc.VectorSubcoreMesh(core_axis_name="core",
                                  subcore_axis_name="subcore")

    @functools.partial(
        pl.kernel,
        out_type=jax.ShapeDtypeStruct((NCORES, N_NODES, D), jnp.float32),
        mesh=mesh,
        scratch_types=[
            pltpu.VMEM((2, CHUNK), jnp.int32),
            pltpu.VMEM((2, CHUNK), jnp.int32),
            pltpu.VMEM((2, CHUNK), jnp.int32),
            pltpu.VMEM((2, CHUNK), jnp.int32),
            pltpu.VMEM((1, CHUNK), jnp.float32),
            pltpu.VMEM((1, CHUNK), jnp.float32),
            pltpu.VMEM((1, CHUNK), jnp.float32),
            pltpu.VMEM((1, CHUNK), jnp.float32),
            pltpu.VMEM((CHUNK, D), jnp.float32),
            pltpu.VMEM((CHUNK, D), jnp.float32),
            pltpu.VMEM_SHARED((N_NODES, D), jnp.float32),
            pltpu.SemaphoreType.DMA,
            pltpu.SemaphoreType.DMA,
            pltpu.SemaphoreType.DMA,
            pltpu.SemaphoreType.DMA,
        ],
    )
    def spmm_kernel(hid_hbm, eidx_hbm, val_hbm, part_hbm,
                    eb0, eb1, eb2, eb3, vb0, vb1, vb2, vb3, rows0, rows1,
                    acc, ssem0, ssem1, isem0, isem1):
        ebufs = (eb0, eb1, eb2, eb3)
        vbufs = (vb0, vb1, vb2, vb3)
        cid = lax.axis_index("core")
        tid = lax.axis_index("subcore")
        wid = tid * NCORES + cid

        @pl.loop(0, RCHUNK)
        def _(r):
            for g in range(D // 16):
                rows0[pl.ds(r, 1), pl.ds(g * 16, 16)] = jnp.zeros(
                    (1, 16), jnp.float32)

        zsrc = rows0.at[pl.ds(0, RCHUNK)]
        for k in range(RITERS):
            rc = k * NSUB + tid

            @pl.when(rc < NRCHUNKS)
            def _():
                pltpu.sync_copy(zsrc, acc.at[pl.ds(rc * RCHUNK, RCHUNK)])
        plsc.subcore_barrier()

        pltpu.sync_copy(eidx_hbm.at[wid], eb0)
        pltpu.sync_copy(val_hbm.at[pl.ds(wid, 1)], vb0)
        pltpu.sync_copy(eidx_hbm.at[NWORKERS + wid], eb1)
        pltpu.sync_copy(val_hbm.at[pl.ds(NWORKERS + wid, 1)], vb1)

        @pl.loop(0, QUADS)
        def _(h):
            for q in range(4):
                rows_v = (rows0, rows1)[q % 2]
                ssem = (ssem0, ssem1)[q % 2]
                isem = (isem0, isem1)[q % 2]
                ebuf = ebufs[q]
                vbuf = vbufs[q]
                ebnx = ebufs[(q + 2) % 4]
                vbnx = vbufs[(q + 2) % 4]
                i = h * 4 + q
                chunk = i * NWORKERS + wid
                nchunk = (i + 2) * NWORKERS + wid

                @pl.when(chunk < NCHUNKS)
                def _():
                    @pl.when(i >= 2)
                    def _():
                        pltpu.make_async_copy(
                            rows_v, acc.at[ebuf.at[1]], ssem).wait()
                        pltpu.make_async_copy(
                            eidx_hbm.at[chunk], ebuf, isem).wait()
                        pltpu.make_async_copy(
                            val_hbm.at[pl.ds(chunk, 1)], vbuf, isem).wait()

                    pltpu.sync_copy(hid_hbm.at[ebuf.at[0]], rows_v)
                    _scale_rows(rows_v, vbuf)
                    pltpu.async_copy(
                        rows_v, acc.at[ebuf.at[1]], ssem, add=True)

                    @pl.when(nchunk < NCHUNKS)
                    def _():
                        pltpu.async_copy(eidx_hbm.at[nchunk], ebnx, isem)
                        pltpu.async_copy(
                            val_hbm.at[pl.ds(nchunk, 1)], vbnx, isem)

        pltpu.make_async_copy(rows0, acc.at[eb0.at[1]], ssem0).wait()
        pltpu.make_async_copy(rows1, acc.at[eb1.at[1]], ssem1).wait()
        plsc.subcore_barrier()

        for k in range(RITERS):
            rc = k * NSUB + tid

            @pl.when(rc < NRCHUNKS)
            def _():
                pltpu.sync_copy(
                    acc.at[pl.ds(rc * RCHUNK, RCHUNK)],
                    part_hbm.at[cid, pl.ds(rc * RCHUNK, RCHUNK)])

    return spmm_kernel(hidden, eidx, vals)


def kernel(input, edge_index, edge_vals, W, b):
    hidden = _matmul(input, W)
    dst = edge_index[0].astype(jnp.int32).reshape(NCHUNKS, CHUNK)
    src = edge_index[1].astype(jnp.int32).reshape(NCHUNKS, CHUNK)
    eidx = jnp.stack([src, dst], axis=1)
    vals = edge_vals.astype(jnp.float32).reshape(NCHUNKS, CHUNK)
    partials = _spmm(hidden, eidx, vals)
    return _combine(partials, b)


# speedup vs baseline: 1.6020x; 1.0024x over previous
import dataclasses
import functools

import jax
import jax.numpy as jnp
from jax import lax
from jax.experimental import pallas as pl
from jax.experimental.pallas import tpu as pltpu
from jax.experimental.pallas import tpu_sc as plsc

N_NODES = 10000
N_EDGES = 320000
D = 128

CHUNK = 128
NCHUNKS = N_EDGES // CHUNK
NCORES = 2
NSUB = 16
NWORKERS = NCORES * NSUB
ITERS = -(-NCHUNKS // NWORKERS)
QUADS = (ITERS + 3) // 4
RCHUNK = 80
NRCHUNKS = N_NODES // RCHUNK
RITERS = -(-NRCHUNKS // NSUB)


def _mm_body(x_ref, w_ref, o_ref):
    o_ref[...] = jnp.dot(x_ref[...], w_ref[...],
                         preferred_element_type=jnp.float32)


def _matmul(x, w):
    return pl.pallas_call(
        _mm_body,
        grid=(10,),
        in_specs=[
            pl.BlockSpec((N_NODES // 10, D), lambda i: (i, 0)),
            pl.BlockSpec((D, D), lambda i: (0, 0)),
        ],
        out_specs=pl.BlockSpec((N_NODES // 10, D), lambda i: (i, 0)),
        out_shape=jax.ShapeDtypeStruct((N_NODES, D), jnp.float32),
    )(x, w)


def _comb_body(p_ref, b_ref, o_ref):
    o_ref[...] = p_ref[0] + p_ref[1] + b_ref[...]


def _combine(partials, b):
    return pl.pallas_call(
        _comb_body,
        grid=(10,),
        in_specs=[
            pl.BlockSpec((2, N_NODES // 10, D), lambda i: (0, i, 0)),
            pl.BlockSpec((1, D), lambda i: (0, 0)),
        ],
        out_specs=pl.BlockSpec((N_NODES // 10, D), lambda i: (i, 0)),
        out_shape=jax.ShapeDtypeStruct((N_NODES, D), jnp.float32),
    )(partials, b)


def _scale_rows(rows_ref, vbuf):
    @pl.loop(0, CHUNK // 16)
    def _(eb):
        v16 = vbuf[pl.ds(0, 1), pl.ds(eb * 16, 16)]
        for j in range(16):
            v = v16[0, j]
            for g in range(D // 16):
                sl = (pl.ds(eb * 16 + j, 1), pl.ds(g * 16, 16))
                rows_ref[sl] = rows_ref[sl] * v


def _spmm(hidden, eidx, vals):
    mesh = plsc.VectorSubcoreMesh(core_axis_name="core",
                                  subcore_axis_name="subcore")

    @functools.partial(
        pl.kernel,
        out_type=jax.ShapeDtypeStruct((NCORES, N_NODES, D), jnp.float32),
        mesh=mesh,
        scratch_types=[
            pltpu.VMEM((2, CHUNK), jnp.int32),
            pltpu.VMEM((2, CHUNK), jnp.int32),
            pltpu.VMEM((2, CHUNK), jnp.int32),
            pltpu.VMEM((2, CHUNK), jnp.int32),
            pltpu.VMEM((1, CHUNK), jnp.float32),
            pltpu.VMEM((1, CHUNK), jnp.float32),
            pltpu.VMEM((1, CHUNK), jnp.float32),
            pltpu.VMEM((1, CHUNK), jnp.float32),
            pltpu.VMEM((CHUNK, D), jnp.float32),
            pltpu.VMEM((CHUNK, D), jnp.float32),
            pltpu.VMEM_SHARED((N_NODES, D), jnp.float32),
            pltpu.SemaphoreType.DMA,
            pltpu.SemaphoreType.DMA,
            pltpu.SemaphoreType.DMA,
            pltpu.SemaphoreType.DMA,
            pltpu.SemaphoreType.DMA,
            pltpu.SemaphoreType.DMA,
        ],
    )
    def spmm_kernel(hid_hbm, eidx_hbm, val_hbm, part_hbm,
                    eb0, eb1, eb2, eb3, vb0, vb1, vb2, vb3, rows0, rows1,
                    acc, ssem0, ssem1, isem0, isem1, gsem0, gsem1):
        ebufs = (eb0, eb1, eb2, eb3)
        vbufs = (vb0, vb1, vb2, vb3)
        cid = lax.axis_index("core")
        tid = lax.axis_index("subcore")
        wid = tid * NCORES + cid

        @pl.loop(0, RCHUNK)
        def _(r):
            for g in range(D // 16):
                rows0[pl.ds(r, 1), pl.ds(g * 16, 16)] = jnp.zeros(
                    (1, 16), jnp.float32)

        zsrc = rows0.at[pl.ds(0, RCHUNK)]
        for k in range(RITERS):
            rc = k * NSUB + tid

            @pl.when(rc < NRCHUNKS)
            def _():
                pltpu.sync_copy(zsrc, acc.at[pl.ds(rc * RCHUNK, RCHUNK)])
        plsc.subcore_barrier()

        pltpu.sync_copy(eidx_hbm.at[wid], eb0)
        pltpu.sync_copy(val_hbm.at[pl.ds(wid, 1)], vb0)
        pltpu.sync_copy(eidx_hbm.at[NWORKERS + wid], eb1)
        pltpu.sync_copy(val_hbm.at[pl.ds(NWORKERS + wid, 1)], vb1)
        pltpu.async_copy(hid_hbm.at[eb0.at[0]], rows0, gsem0)

        @pl.loop(0, QUADS)
        def _(h):
            for q in range(4):
                rows_v = (rows0, rows1)[q % 2]
                ssem = (ssem0, ssem1)[q % 2]
                gsem = (gsem0, gsem1)[q % 2]
                rwn = (rows0, rows1)[(q + 1) % 2]
                ssn = (ssem0, ssem1)[(q + 1) % 2]
                gsn = (gsem0, gsem1)[(q + 1) % 2]
                isn = (isem0, isem1)[(q + 1) % 2]
                isi = (isem0, isem1)[q % 2]
                ebuf = ebufs[q]
                vbuf = vbufs[q]
                ebn = ebufs[(q + 1) % 4]
                vbn = vbufs[(q + 1) % 4]
                ebnn = ebufs[(q + 2) % 4]
                vbnn = vbufs[(q + 2) % 4]
                i = h * 4 + q
                chunk = i * NWORKERS + wid
                nchunk = (i + 1) * NWORKERS + wid
                nnchunk = (i + 2) * NWORKERS + wid

                @pl.when(chunk < NCHUNKS)
                def _():
                    pltpu.make_async_copy(
                        hid_hbm.at[ebuf.at[0]], rows_v, gsem).wait()
                    _scale_rows(rows_v, vbuf)
                    pltpu.async_copy(
                        rows_v, acc.at[ebuf.at[1]], ssem, add=True)

                @pl.when(nchunk < NCHUNKS)
                def _():
                    @pl.when(i >= 1)
                    def _():
                        pltpu.make_async_copy(
                            rwn, acc.at[ebn.at[1]], ssn).wait()

                    @pl.when(i + 1 >= 2)
                    def _():
                        pltpu.make_async_copy(
                            eidx_hbm.at[nchunk], ebn, isn).wait()
                        pltpu.make_async_copy(
                            val_hbm.at[pl.ds(nchunk, 1)], vbn, isn).wait()

                    pltpu.async_copy(hid_hbm.at[ebn.at[0]], rwn, gsn)

                    @pl.when(nnchunk < NCHUNKS)
                    def _():
                        pltpu.async_copy(eidx_hbm.at[nnchunk], ebnn, isi)
                        pltpu.async_copy(
                            val_hbm.at[pl.ds(nnchunk, 1)], vbnn, isi)

        pltpu.make_async_copy(rows0, acc.at[eb0.at[1]], ssem0).wait()
        pltpu.make_async_copy(rows1, acc.at[eb1.at[1]], ssem1).wait()
        plsc.subcore_barrier()

        for k in range(RITERS):
            rc = k * NSUB + tid

            @pl.when(rc < NRCHUNKS)
            def _():
                pltpu.sync_copy(
                    acc.at[pl.ds(rc * RCHUNK, RCHUNK)],
                    part_hbm.at[cid, pl.ds(rc * RCHUNK, RCHUNK)])

    return spmm_kernel(hidden, eidx, vals)


def kernel(input, edge_index, edge_vals, W, b):
    hidden = _matmul(input, W)
    dst = edge_index[0].astype(jnp.int32).reshape(NCHUNKS, CHUNK)
    src = edge_index[1].astype(jnp.int32).reshape(NCHUNKS, CHUNK)
    eidx = jnp.stack([src, dst], axis=1)
    vals = edge_vals.astype(jnp.float32).reshape(NCHUNKS, CHUNK)
    partials = _spmm(hidden, eidx, vals)
    return _combine(partials, b)
